# Initial kernel scaffold; baseline (speedup 1.0000x reference)
import jax
import jax.numpy as jnp
from jax.experimental import pallas as pl


def kernel(click_vals, favor_vals, consume_vals, edge_rows, edge_cols, uid_embed, pid_embed, user_index, poi_index, labels, global_user_feature, w, fw, W_cu, W_cp, W_fu, W_fp, W_su, W_sp, W_sel, b_sel):
    raise NotImplementedError("write your pallas kernel here")


# speedup vs baseline: 1.0520x; 1.0520x over previous
import functools

import jax
import jax.numpy as jnp
from jax.experimental import pallas as pl
from jax.experimental.pallas import tpu as pltpu

N_USERS = 10000
N_POIS = 10000
D = 128
TOPN = 64
_R1 = 1000


def _norm_rows(x):
    return x * (1.0 / jnp.sqrt(jnp.sum(x * x, axis=1, keepdims=True)))


def _bf(x):
    return x.astype(jnp.bfloat16).astype(jnp.float32)


def _dot(a, b):
    return jax.lax.dot_general(a, b, (((1,), (1,)), ((), ())),
                               preferred_element_type=jnp.float32)


def _t1_body(u0, l1f, l2f, l1s, l2s, l1c, l2c, wf, ws, wc, fwb, out):
    fw = fwb[...]
    acc = None
    for i, (l1, l2, w) in enumerate(((l1f, l2f, wf), (l1s, l2s, ws),
                                     (l1c, l2c, wc))):
        all_u = u0[...] + _norm_rows(l1[...]) + _norm_rows(l2[...])
        uf = _dot(all_u, w[...])
        part = _bf(uf) * fw[i:i + 1, :]
        acc = part if acc is None else acc + part
    out[...] = acc


def _t1(u0, l1f, l2f, l1s, l2s, l1c, l2c, wf, ws, wc, fwb):
    nb = N_USERS // _R1
    row = pl.BlockSpec((_R1, D), lambda i: (i, 0))
    full = pl.BlockSpec((D, D), lambda i: (0, 0))
    fwspec = pl.BlockSpec((3, D), lambda i: (0, 0))
    return pl.pallas_call(
        _t1_body,
        grid=(nb,),
        in_specs=[row] * 7 + [full] * 3 + [fwspec],
        out_specs=row,
        out_shape=jax.ShapeDtypeStruct((N_USERS, D), jnp.float32),
    )(u0, l1f, l2f, l1s, l2s, l1c, l2c, wf, ws, wc, fwb)


def _t2_body(bsf, bid, g, out):
    uu = jax.lax.dot_general(bsf[...], bid[...], (((1,), (1,)), ((), ())),
                             preferred_element_type=jnp.float32)
    imin = jnp.int32(-(2**31))
    u = jax.lax.bitcast_convert_type(uu, jnp.int32)
    kb = jnp.where(u < 0, ~u, u | imin)
    ks = kb ^ imin
    tb = jnp.zeros((bsf.shape[0], 1), jnp.int32)
    for b in range(31, -1, -1):
        cb = tb | jnp.int32((1 << b) - 2**32 if b == 31 else 1 << b)
        cs = cb ^ imin
        cnt = jnp.sum((ks >= cs).astype(jnp.float32), axis=1, keepdims=True)
        tb = jnp.where(cnt >= TOPN, cb, tb)
    ts = tb ^ imin
    cnt_ge = jnp.sum((ks >= ts).astype(jnp.float32), axis=1, keepdims=True)
    cnt_gt = jnp.sum((ks > ts).astype(jnp.float32), axis=1, keepdims=True)
    factor = (TOPN - cnt_gt) / jnp.maximum(cnt_ge - cnt_gt, 1.0)
    m = jnp.max(uu, axis=1, keepdims=True)
    e = jnp.exp(uu - m)
    wt = jnp.where(ks > ts, e, jnp.where(ks == ts, e * factor, 0.0))
    wt = wt / jnp.sum(wt, axis=1, keepdims=True)
    out[...] = jnp.dot(wt, g[...], preferred_element_type=jnp.float32)


def _t2(bs_feat, behavior_id, g):
    return pl.pallas_call(
        _t2_body,
        out_shape=jax.ShapeDtypeStruct((bs_feat.shape[0], D), jnp.float32),
    )(bs_feat, behavior_id, g)


def _t3a_body(ub0, lb1f, lb2f, lb1s, lb2s, lb1c, lb2c,
              wuf, wus, wuc, wpf, wps, wpc, wsel, bsel, fwb, wb,
              bsf_out, qf_out, qs_out, qc_out):
    ufb = []
    for l1, l2, wu in ((lb1f, lb2f, wuf), (lb1s, lb2s, wus), (lb1c, lb2c, wuc)):
        all_ub = ub0[...] + _norm_rows(l1[...]) + _norm_rows(l2[...])
        ufb.append(jax.lax.dot_general(all_ub, wu[...], (((1,), (1,)), ((), ())),
                                       preferred_element_type=jnp.float32))
    fw = fwb[...]
    bs_id = (_bf(ufb[0]) * fw[0:1, :] + _bf(ufb[1]) * fw[1:2, :]
             + _bf(ufb[2]) * fw[2:3, :])
    bsf_out[...] = jax.lax.dot_general(
        bs_id, wsel[...], (((1,), (1,)), ((), ())),
        preferred_element_type=jnp.float32) + bsel[...]
    wv = wb[...]
    for i, (wp, wrow, out) in enumerate(((wpf, wv[0:1, :], qf_out),
                                         (wps, wv[2:3, :], qs_out),
                                         (wpc, wv[1:2, :], qc_out))):
        out[...] = jnp.dot(ufb[i], wp[...],
                           preferred_element_type=jnp.float32) * wrow


def _t3a(ub0, lbs, wu3, wp3, wsel, bsel, fwb, wb):
    B = ub0.shape[0]
    sh = jax.ShapeDtypeStruct((B, D), jnp.float32)
    return pl.pallas_call(
        _t3a_body,
        out_shape=(sh, sh, sh, sh),
    )(ub0, *lbs, *wu3, *wp3, wsel, bsel, fwb, wb)


_BB = 64


def _t3b_body(qf, qs, qc, pb0, pb1f, pb2f, pb1s, pb2s, pb1c, pb2c,
              labels, loss_out):
    nb = _BB
    alt = None
    for q, p1, p2 in ((qf, pb1f, pb2f), (qs, pb1s, pb2s), (qc, pb1c, pb2c)):
        all_pb = pb0[...] + _norm_rows(p1[...]) + _norm_rows(p2[...])
        neg = all_pb.shape[0] // nb
        prod = all_pb.reshape(nb, neg, D) * q[...][:, None, :]
        part = jnp.sum(prod, axis=2)
        alt = part if alt is None else alt + part
    y = labels[...]
    ll = jnp.maximum(alt, 0.0) - alt * y + jnp.log1p(jnp.exp(-jnp.abs(alt)))
    s = jnp.sum(ll).reshape(1, 1)

    @pl.when(pl.program_id(0) == 0)
    def _():
        loss_out[...] = jnp.zeros_like(loss_out)

    loss_out[...] += s


def _t3b(q3, pbs, labels):
    B = labels.shape[0]
    neg = labels.shape[1]
    nblk = B // _BB
    qspec = pl.BlockSpec((_BB, D), lambda i: (i, 0))
    pspec = pl.BlockSpec((_BB * neg, D), lambda i: (i, 0))
    lspec = pl.BlockSpec((_BB, neg), lambda i: (i, 0))
    loss = pl.pallas_call(
        _t3b_body,
        grid=(nblk,),
        in_specs=[qspec] * 3 + [pspec] * 7 + [lspec],
        out_specs=pl.BlockSpec((1, 1), lambda i: (0, 0)),
        out_shape=jax.ShapeDtypeStruct((1, 1), jnp.float32),
    )(*q3, *pbs, labels)
    return loss / (B * neg)


def _spmm(vals, rows, cols, x, n_out):
    return jax.ops.segment_sum(vals[:, None] * x[cols], rows, num_segments=n_out)


def kernel(click_vals, favor_vals, consume_vals, edge_rows, edge_cols,
           uid_embed, pid_embed, user_index, poi_index, labels,
           global_user_feature, w, fw, W_cu, W_cp, W_fu, W_fp, W_su, W_sp,
           W_sel, b_sel):
    B = user_index.shape[0]
    batch_user = user_index.reshape(-1)
    flat_poi = poi_index.reshape(-1)
    w_s = jax.nn.softmax(w, axis=1)
    fw_s = jax.nn.softmax(fw, axis=0)

    favor_v = favor_vals + 1e-18 * click_vals
    consume_v = consume_vals + 1e-18 * click_vals

    lus, lps = {}, {}
    for name, vals in (("f", favor_v), ("s", consume_v), ("c", click_vals)):
        lu1 = _spmm(vals, edge_rows, edge_cols, pid_embed, N_USERS)
        lp1 = _spmm(vals, edge_cols, edge_rows, lu1, N_POIS)
        lu2 = _spmm(vals, edge_rows, edge_cols, lp1, N_USERS)
        lp2 = _spmm(vals, edge_cols, edge_rows, lu2, N_POIS)
        lus[name] = (lu1, lu2)
        lps[name] = (lp1, lp2)

    fwb = jnp.broadcast_to(
        fw_s.reshape(3, 1).astype(jnp.bfloat16).astype(jnp.float32), (3, D))
    behavior_id = _t1(uid_embed, lus["f"][0], lus["f"][1], lus["s"][0],
                      lus["s"][1], lus["c"][0], lus["c"][1],
                      W_fu, W_su, W_cu, fwb)

    ub0 = uid_embed[batch_user]
    lbs = [lus["f"][0][batch_user], lus["f"][1][batch_user],
           lus["s"][0][batch_user], lus["s"][1][batch_user],
           lus["c"][0][batch_user], lus["c"][1][batch_user]]
    pbs = [pid_embed[flat_poi],
           lps["f"][0][flat_poi], lps["f"][1][flat_poi],
           lps["s"][0][flat_poi], lps["s"][1][flat_poi],
           lps["c"][0][flat_poi], lps["c"][1][flat_poi]]
    wb = jnp.broadcast_to(w_s.reshape(3, 1), (3, D))
    labels_r = labels.reshape(B, -1)
    bs_feat, qf, qs, qc = _t3a(ub0, lbs, (W_fu, W_su, W_cu),
                               (W_fp, W_sp, W_cp), W_sel,
                               b_sel.reshape(1, D), fwb, wb)
    loss = _t3b((qf, qs, qc), pbs, labels_r)
    user_feature = _t2(bs_feat, behavior_id, global_user_feature)
    return loss.reshape(()), user_feature
